# baseline (speedup 1.0000x reference)
import jax
import jax.numpy as jnp
from jax.experimental import pallas as pl
from jax.experimental.pallas import tpu as pltpu

B = 4
E = 16
C = 1024
K = E // 2
NB = 8
BC = C // NB


def _gate(x):
    rs = jnp.sum(x, axis=-1)
    acc = jnp.zeros((B,), jnp.float32)
    cur = rs
    iota = jax.lax.broadcasted_iota(jnp.int32, (B, E), 1)
    for _ in range(K):
        m = jnp.max(cur, axis=1)
        acc = acc + m
        is_max = cur == m[:, None]
        first = jnp.min(jnp.where(is_max, iota, E), axis=1)
        cur = jnp.where(iota == first[:, None], -jnp.inf, cur)
    return acc * (1.0 / K)


def _moe_kernel(x_ref, wv_ref, wo_ref, out_ref, v_ref):
    i = pl.program_id(0)

    @pl.when(i == 0)
    def _():
        v_ref[...] = jnp.zeros_like(v_ref)
        out_ref[...] = jnp.zeros_like(out_ref)

    @pl.when(i < NB)
    def _():
        x0_blk = x_ref[:, 0, pl.ds(i * BC, BC)]
        v_ref[...] += jnp.dot(x0_blk, wv_ref[0],
                              preferred_element_type=jnp.float32)

    @pl.when(i >= NB)
    def _():
        j = i - NB
        v_blk = v_ref[:, pl.ds(j * BC, BC)]
        out_ref[...] += jnp.dot(v_blk, wo_ref[0],
                                preferred_element_type=jnp.float32)

    @pl.when(i == 2 * NB - 1)
    def _():
        out_ref[...] *= _gate(x_ref[...])[:, None]


def kernel(x, q, Wq, Wk, Wv, Wo):
    out = pl.pallas_call(
        _moe_kernel,
        grid=(2 * NB,),
        in_specs=[
            pl.BlockSpec((B, E, C), lambda i: (0, 0, 0)),
            pl.BlockSpec((1, BC, C), lambda i: (0, jnp.minimum(i, NB - 1), 0)),
            pl.BlockSpec((1, BC, C), lambda i: (0, jnp.maximum(i - NB, 0), 0)),
        ],
        out_specs=pl.BlockSpec((B, C), lambda i: (0, 0)),
        out_shape=jax.ShapeDtypeStruct((B, C), jnp.float32),
        scratch_shapes=[pltpu.VMEM((B, C), jnp.float32)],
    )(x, Wv, Wo)
    return out[:, None, :]


# speedup vs baseline: 1.6677x; 1.6677x over previous
import jax
import jax.numpy as jnp
from jax.experimental import pallas as pl

B = 4
E = 16
C = 1024
K = E // 2
NS = 4
BC = 128


def _gate(x):
    rs = jnp.sum(x, axis=-1)
    acc = jnp.zeros((B,), jnp.float32)
    cur = rs
    iota = jax.lax.broadcasted_iota(jnp.int32, (B, E), 1)
    for _ in range(K):
        m = jnp.max(cur, axis=1)
        acc = acc + m
        is_max = cur == m[:, None]
        first = jnp.min(jnp.where(is_max, iota, E), axis=1)
        cur = jnp.where(iota == first[:, None], -jnp.inf, cur)
    return acc * (1.0 / K)


def _moe_kernel(x_ref, wv1_ref, wv2_ref, wo1_ref, wo2_ref, out_ref):
    i = pl.program_id(0)
    x0 = x_ref[:, 0, :]
    v1 = jnp.dot(x0, wv1_ref[0], preferred_element_type=jnp.float32)
    v2 = jnp.dot(x0, wv2_ref[0], preferred_element_type=jnp.float32)
    contrib = (jnp.dot(v1, wo1_ref[0], preferred_element_type=jnp.float32)
               + jnp.dot(v2, wo2_ref[0], preferred_element_type=jnp.float32))

    @pl.when(i == 0)
    def _():
        out_ref[...] = jnp.zeros_like(out_ref)

    @pl.when(i < NS - 1)
    def _():
        out_ref[...] += contrib

    @pl.when(i == NS - 1)
    def _():
        out_ref[...] = (out_ref[...] + contrib) * _gate(x_ref[...])[:, None]


def kernel(x, q, Wq, Wk, Wv, Wo):
    out = pl.pallas_call(
        _moe_kernel,
        grid=(NS,),
        in_specs=[
            pl.BlockSpec((B, E, C), lambda i: (0, 0, 0)),
            pl.BlockSpec((1, C, BC), lambda i: (0, 0, i)),
            pl.BlockSpec((1, C, BC), lambda i: (0, 0, i + NS)),
            pl.BlockSpec((1, BC, C), lambda i: (0, i, 0)),
            pl.BlockSpec((1, BC, C), lambda i: (0, i + NS, 0)),
        ],
        out_specs=pl.BlockSpec((B, C), lambda i: (0, 0)),
        out_shape=jax.ShapeDtypeStruct((B, C), jnp.float32),
    )(x, Wv, Wv, Wo, Wo)
    return out[:, None, :]


# speedup vs baseline: 1.7538x; 1.0517x over previous
import jax
import jax.numpy as jnp
from jax.experimental import pallas as pl

B = 4
E = 16
C = 1024
K = E // 2
NS = 2
NSTR = 4
BC = 128


def _gate(x):
    rs = jnp.sum(x, axis=-1)
    acc = jnp.zeros((B,), jnp.float32)
    cur = rs
    iota = jax.lax.broadcasted_iota(jnp.int32, (B, E), 1)
    for _ in range(K):
        m = jnp.max(cur, axis=1)
        acc = acc + m
        is_max = cur == m[:, None]
        first = jnp.min(jnp.where(is_max, iota, E), axis=1)
        cur = jnp.where(iota == first[:, None], -jnp.inf, cur)
    return acc * (1.0 / K)


def _moe_kernel(x_ref, *refs):
    wv_refs = refs[:NSTR]
    wo_refs = refs[NSTR:2 * NSTR]
    out_ref = refs[2 * NSTR]
    i = pl.program_id(0)
    x0 = x_ref[:, 0, :]
    contrib = jnp.zeros((B, C), jnp.float32)
    for s in range(NSTR):
        v = jnp.dot(x0, wv_refs[s][0], preferred_element_type=jnp.float32)
        contrib += jnp.dot(v, wo_refs[s][0],
                           preferred_element_type=jnp.float32)

    @pl.when(i == 0)
    def _():
        out_ref[...] = jnp.zeros_like(out_ref)

    @pl.when(i < NS - 1)
    def _():
        out_ref[...] += contrib

    @pl.when(i == NS - 1)
    def _():
        out_ref[...] = (out_ref[...] + contrib) * _gate(x_ref[...])[:, None]


def kernel(x, q, Wq, Wk, Wv, Wo):
    def wv_spec(s):
        return pl.BlockSpec((1, C, BC), lambda i, s=s: (0, 0, s * NS + i))

    def wo_spec(s):
        return pl.BlockSpec((1, BC, C), lambda i, s=s: (0, s * NS + i, 0))

    out = pl.pallas_call(
        _moe_kernel,
        grid=(NS,),
        in_specs=[pl.BlockSpec((B, E, C), lambda i: (0, 0, 0))]
        + [wv_spec(s) for s in range(NSTR)]
        + [wo_spec(s) for s in range(NSTR)],
        out_specs=pl.BlockSpec((B, C), lambda i: (0, 0)),
        out_shape=jax.ShapeDtypeStruct((B, C), jnp.float32),
    )(x, *([Wv] * NSTR), *([Wo] * NSTR))
    return out[:, None, :]
